# baseline (device time: 33900 ns/iter reference)
import jax
import jax.numpy as jnp
from jax import lax
from jax.experimental import pallas as pl
from jax.experimental.pallas import tpu as pltpu

M = 1024
N = 1024
H = M // 2
K_COM = 16
R = H // K_COM
CHUNKS = ((0, 256), (256, 256))


def kernel(dy, W):
    def body(dy_ref, w_ref, out_ref, pbuf, ybuf,
             ysend_sems, yrecv_sems, xsend_sems, xrecv_sems):
        my_x = lax.axis_index("x")
        my_y = lax.axis_index("y")

        barrier_sem = pltpu.get_barrier_semaphore()
        pl.semaphore_signal(
            barrier_sem, inc=1,
            device_id=(my_x, 1 - my_y), device_id_type=pl.DeviceIdType.MESH)
        pl.semaphore_signal(
            barrier_sem, inc=1,
            device_id=(1 - my_x, my_y), device_id_type=pl.DeviceIdType.MESH)

        row0 = my_x * H

        def y_copy(k):
            return pltpu.make_async_remote_copy(
                src_ref=pbuf.at[pl.ds(k * R, R)],
                dst_ref=ybuf.at[pl.ds(k * R, R)],
                send_sem=ysend_sems.at[k],
                recv_sem=yrecv_sems.at[k],
                device_id=(my_x, 1 - my_y),
                device_id_type=pl.DeviceIdType.MESH,
            )

        def x_copy(k):
            return pltpu.make_async_remote_copy(
                src_ref=out_ref.at[pl.ds(row0 + k * R, R)],
                dst_ref=out_ref.at[pl.ds(row0 + k * R, R)],
                send_sem=xsend_sems.at[k],
                recv_sem=xrecv_sems.at[k],
                device_id=(1 - my_x, my_y),
                device_id_type=pl.DeviceIdType.MESH,
            )

        kstart = 0
        for r0, rl in CHUNKS:
            a = dy_ref[pl.ds(row0 + r0, rl), :]
            p = lax.dot_general(
                a, w_ref[...],
                dimension_numbers=(((1,), (1,)), ((), ())),
                preferred_element_type=jnp.float32,
            )
            pbuf[pl.ds(r0, rl), :] = p.astype(jnp.bfloat16)
            if r0 == 0:
                pl.semaphore_wait(barrier_sem, 2)
            for s in range(rl // R):
                y_copy(kstart + s).start()
            kstart += rl // R

        for k in range(K_COM):
            yc = y_copy(k)
            yc.wait_send()
            yc.wait_recv()
            out_ref[pl.ds(row0 + k * R, R), :] = (
                pbuf[pl.ds(k * R, R), :] + ybuf[pl.ds(k * R, R), :]
            )
            x_copy(k).start()

        for k in range(K_COM):
            x_copy(k).wait()

    return pl.pallas_call(
        body,
        out_shape=jax.ShapeDtypeStruct((M, N), jnp.bfloat16),
        in_specs=[
            pl.BlockSpec(memory_space=pltpu.VMEM),
            pl.BlockSpec(memory_space=pltpu.VMEM),
        ],
        out_specs=pl.BlockSpec(memory_space=pltpu.VMEM),
        scratch_shapes=[
            pltpu.VMEM((H, N), jnp.bfloat16),
            pltpu.VMEM((H, N), jnp.bfloat16),
            pltpu.SemaphoreType.DMA((K_COM,)),
            pltpu.SemaphoreType.DMA((K_COM,)),
            pltpu.SemaphoreType.DMA((K_COM,)),
            pltpu.SemaphoreType.DMA((K_COM,)),
        ],
        compiler_params=pltpu.CompilerParams(collective_id=0),
    )(dy, W)
